# baseline (speedup 1.0000x reference)
import functools

import jax
import jax.numpy as jnp
from jax.experimental import pallas as pl
from jax.experimental.pallas import tpu as pltpu

N_FEATURES = 4096
IN_FEATURES = 4096
OUT_FEATURES = 4096
BATCH = 4096
REPLACE_RATE = 1e-4
DECAY_RATE = 0.99
MATURITY_THRESHOLD = 100

_BR = 256
_K = 8


def _stats_body(wout_ref, x_ref, win_ref, wout_copy_ref, win_copy_ref,
                wsum_ref, xsum_ref):
    i = pl.program_id(0)
    wout = wout_ref[...]
    x = x_ref[...]
    wpart = jnp.sum(jnp.abs(wout), axis=0, keepdims=True)
    xpart = jnp.sum(jnp.abs(x), axis=0, keepdims=True)

    @pl.when(i == 0)
    def _():
        wsum_ref[...] = wpart
        xsum_ref[...] = xpart

    @pl.when(i != 0)
    def _():
        wsum_ref[...] += wpart
        xsum_ref[...] += xpart

    wout_copy_ref[...] = wout
    win_copy_ref[...] = win_ref[...]


def _stats_pass(weights_out, input_values, weights_in):
    grid = (OUT_FEATURES // _BR,)
    return pl.pallas_call(
        _stats_body,
        grid=grid,
        in_specs=[
            pl.BlockSpec((_BR, N_FEATURES), lambda i: (i, 0)),
            pl.BlockSpec((_BR, N_FEATURES), lambda i: (i, 0)),
            pl.BlockSpec((_BR, N_FEATURES), lambda i: (i, 0)),
        ],
        out_specs=[
            pl.BlockSpec((_BR, N_FEATURES), lambda i: (i, 0)),
            pl.BlockSpec((_BR, N_FEATURES), lambda i: (i, 0)),
            pl.BlockSpec((1, N_FEATURES), lambda i: (0, 0)),
            pl.BlockSpec((1, N_FEATURES), lambda i: (0, 0)),
        ],
        out_shape=[
            jax.ShapeDtypeStruct((OUT_FEATURES, N_FEATURES), jnp.float32),
            jax.ShapeDtypeStruct((N_FEATURES, IN_FEATURES), jnp.float32),
            jax.ShapeDtypeStruct((1, N_FEATURES), jnp.float32),
            jax.ShapeDtypeStruct((1, N_FEATURES), jnp.float32),
        ],
    )(weights_out, input_values, weights_in)


def _order_stat_bits(bits, k):
    def body(_, carry):
        lo, hi = carry
        mid = jax.lax.div(lo + hi, 2)
        cnt = jnp.sum((bits <= mid).astype(jnp.int32))
        big = cnt >= k
        return jnp.where(big, lo, mid + 1), jnp.where(big, mid, hi)

    lo = jnp.int32(0)
    hi = jnp.max(bits)
    lo, hi = jax.lax.fori_loop(0, 31, body, (lo, hi))
    return hi


def _rank_body(wsum_ref, xsum_ref, util_ref, age_ref, noise_ref, acc_ref,
               util3_ref, age3_ref, mask_ref, nrepl_ref, sidx_ref, svalid_ref):
    wsum = wsum_ref[...]
    xsum = xsum_ref[...]
    utility = util_ref[...]
    age = age_ref[...]
    noise = noise_ref[...]

    age2 = age + 1
    input_magnitudes = xsum * jnp.float32(1.0 / BATCH)
    step_utility = input_magnitudes * wsum
    utility2 = (jnp.float32(1.0 - DECAY_RATE) * step_utility
                + jnp.float32(DECAY_RATE) * utility)

    acc2 = acc_ref[0] + jnp.float32(REPLACE_RATE * N_FEATURES)
    n_available = acc2.astype(jnp.int32)

    eligibility = age2 > MATURITY_THRESHOLD
    n_eligible = jnp.sum(eligibility.astype(jnp.int32))
    n_repl = jnp.where(n_available > 0,
                       jnp.minimum(n_available, n_eligible),
                       0)

    perturbed = utility2 + noise
    inf = jnp.float32(jnp.inf)
    filtered = jnp.where(eligibility, perturbed, inf)

    m1 = jnp.min(filtered)
    c1 = jnp.sum((filtered == m1).astype(jnp.int32))
    m2 = jnp.min(jnp.where(filtered > m1, filtered, inf))
    threshold = jnp.where(c1 >= n_repl, m1, m2)

    prune = (filtered <= threshold) & eligibility & (n_repl > 0)

    bits = jax.lax.bitcast_convert_type(utility2, jnp.int32)
    half = N_FEATURES // 2
    b_lo = _order_stat_bits(bits, half)
    b_hi = _order_stat_bits(bits, half + 1)
    s_lo = jax.lax.bitcast_convert_type(b_lo, jnp.float32)
    s_hi = jax.lax.bitcast_convert_type(b_hi, jnp.float32)
    median = (s_lo + s_hi) * jnp.float32(0.5)

    util3_ref[...] = jnp.where(prune, median, utility2)
    age3_ref[...] = jnp.where(prune, 0, age2)
    mask_ref[...] = prune.astype(jnp.int32)
    nrepl_ref[0] = n_repl

    positions = jax.lax.broadcasted_iota(jnp.int32, (1, N_FEATURES), 1)
    big = jnp.int32(N_FEATURES)
    pad = jnp.min(jnp.where(prune, big, positions))

    def slot(k, posm):
        m = jnp.min(posm)
        found = m < big
        sidx_ref[k] = jnp.where(found, m, pad)
        svalid_ref[k] = found.astype(jnp.int32)
        return jnp.where(posm == m, big, posm)

    jax.lax.fori_loop(0, _K, slot, jnp.where(prune, positions, big))


def _rank_pass(wsum, xsum, utility, age, noise, acc_in):
    outs = pl.pallas_call(
        _rank_body,
        in_specs=[
            pl.BlockSpec((1, N_FEATURES), lambda: (0, 0)),
            pl.BlockSpec((1, N_FEATURES), lambda: (0, 0)),
            pl.BlockSpec((1, N_FEATURES), lambda: (0, 0)),
            pl.BlockSpec((1, N_FEATURES), lambda: (0, 0)),
            pl.BlockSpec((1, N_FEATURES), lambda: (0, 0)),
            pl.BlockSpec(memory_space=pltpu.SMEM),
        ],
        out_specs=[
            pl.BlockSpec((1, N_FEATURES), lambda: (0, 0)),
            pl.BlockSpec((1, N_FEATURES), lambda: (0, 0)),
            pl.BlockSpec((1, N_FEATURES), lambda: (0, 0)),
            pl.BlockSpec(memory_space=pltpu.SMEM),
            pl.BlockSpec(memory_space=pltpu.SMEM),
            pl.BlockSpec(memory_space=pltpu.SMEM),
        ],
        out_shape=[
            jax.ShapeDtypeStruct((1, N_FEATURES), jnp.float32),
            jax.ShapeDtypeStruct((1, N_FEATURES), jnp.int32),
            jax.ShapeDtypeStruct((1, N_FEATURES), jnp.int32),
            jax.ShapeDtypeStruct((1,), jnp.int32),
            jax.ShapeDtypeStruct((_K,), jnp.int32),
            jax.ShapeDtypeStruct((_K,), jnp.int32),
        ],
    )(wsum, xsum, utility, age, noise, acc_in)
    return outs


_LECUN_LIMIT = 0.027063293382525444


def _rotl(x, r):
    return jax.lax.shift_left(x, jnp.int32(r)) | jax.lax.shift_right_logical(
        x, jnp.int32(32 - r))


def _threefry_round(x0, x1, r):
    x0 = x0 + x1
    x1 = _rotl(x1, r)
    x1 = x0 ^ x1
    return x0, x1


def _lecun_row_bits(row, k0, k1):
    ks2 = k0 ^ k1 ^ jnp.int32(0x1BD11BDA)
    lanes = jax.lax.broadcasted_iota(jnp.int32, (1, IN_FEATURES), 1)
    x0 = jnp.zeros((1, IN_FEATURES), jnp.int32) + k0
    x1 = (row * jnp.int32(IN_FEATURES) + lanes) + k1

    r1 = (13, 15, 26, 6)
    r2 = (17, 29, 16, 24)
    for r in r1:
        x0, x1 = _threefry_round(x0, x1, r)
    x0 = x0 + k1
    x1 = x1 + ks2 + jnp.int32(1)
    for r in r2:
        x0, x1 = _threefry_round(x0, x1, r)
    x0 = x0 + ks2
    x1 = x1 + k0 + jnp.int32(2)
    for r in r1:
        x0, x1 = _threefry_round(x0, x1, r)
    x0 = x0 + k0
    x1 = x1 + k1 + jnp.int32(3)
    for r in r2:
        x0, x1 = _threefry_round(x0, x1, r)
    x0 = x0 + k1
    x1 = x1 + ks2 + jnp.int32(4)
    for r in r1:
        x0, x1 = _threefry_round(x0, x1, r)
    x0 = x0 + ks2
    x1 = x1 + k0 + jnp.int32(5)

    bits = x0 ^ x1
    float_bits = jax.lax.shift_right_logical(bits, jnp.int32(9)) | jnp.int32(
        0x3F800000)
    floats = jax.lax.bitcast_convert_type(float_bits, jnp.float32) - jnp.float32(1.0)
    mn = jnp.float32(-_LECUN_LIMIT)
    mx = jnp.float32(_LECUN_LIMIT)
    return jnp.maximum(mn, floats * (mx - mn) + mn)


def _scatter_body(sidx_ref, svalid_ref, skey_ref, winrow_ref, maskg_ref,
                  woutg_ref, win_out_ref, wout_out_ref):
    i = pl.program_id(0)
    valid = svalid_ref[i] != 0
    row = sidx_ref[i]
    newrow = _lecun_row_bits(row, skey_ref[0], skey_ref[1]).reshape(
        1, 1, IN_FEATURES)
    win_out_ref[...] = jnp.where(valid, newrow, winrow_ref[...])
    m = maskg_ref[...].reshape(1, 128) != 0
    wout_out_ref[...] = jnp.where(m, jnp.float32(0.0), woutg_ref[...])


def _scatter_pass(sidx, svalid, skey, win3, mask3, wout_copy):
    grid_spec = pltpu.PrefetchScalarGridSpec(
        num_scalar_prefetch=3,
        grid=(_K,),
        in_specs=[
            pl.BlockSpec((1, 1, IN_FEATURES),
                         lambda i, s, v, kd: (s[i], 0, 0)),
            pl.BlockSpec((1, 1, 128),
                         lambda i, s, v, kd: (s[i] // 128, 0, 0)),
            pl.BlockSpec((OUT_FEATURES, 128),
                         lambda i, s, v, kd: (0, s[i] // 128)),
        ],
        out_specs=[
            pl.BlockSpec((1, 1, IN_FEATURES),
                         lambda i, s, v, kd: (s[i], 0, 0)),
            pl.BlockSpec((OUT_FEATURES, 128),
                         lambda i, s, v, kd: (0, s[i] // 128)),
        ],
    )
    return pl.pallas_call(
        _scatter_body,
        grid_spec=grid_spec,
        out_shape=[
            jax.ShapeDtypeStruct((N_FEATURES, 1, IN_FEATURES), jnp.float32),
            jax.ShapeDtypeStruct((OUT_FEATURES, N_FEATURES), jnp.float32),
        ],
        input_output_aliases={3: 0, 5: 1},
    )(sidx, svalid, skey, win3, mask3, wout_copy)


def kernel(weights_in, weights_out, input_values, age, utility,
           replacement_accumulator):
    noise_key, in_key = jax.random.split(jax.random.key(42))
    noise = jax.random.normal(noise_key, (N_FEATURES,)) * 1e-12
    skey = jax.random.key_data(in_key).astype(jnp.uint32).view(jnp.int32)

    wout_copy, win_copy, wsum, xsum = _stats_pass(
        weights_out, input_values, weights_in)

    acc2 = replacement_accumulator + REPLACE_RATE * N_FEATURES
    util3, age3, mask_i32, nrepl, sidx, svalid = _rank_pass(
        wsum, xsum,
        utility.reshape(1, N_FEATURES),
        age.reshape(1, N_FEATURES),
        noise.reshape(1, N_FEATURES),
        replacement_accumulator.reshape(1),
    )
    mask_flat = mask_i32.reshape(N_FEATURES)
    prune_mask = mask_flat.astype(jnp.bool_)
    n_repl = nrepl[0]

    win2_3d, weights_out2 = _scatter_pass(
        sidx, svalid, skey,
        win_copy.reshape(N_FEATURES, 1, IN_FEATURES),
        mask_i32.reshape(N_FEATURES // 128, 1, 128),
        wout_copy)
    weights_in2 = win2_3d.reshape(N_FEATURES, IN_FEATURES)

    utility3 = util3.reshape(N_FEATURES)
    age3 = age3.reshape(N_FEATURES)
    acc3 = acc2 - n_repl.astype(jnp.float32)

    return (utility3, age3, prune_mask, weights_in2, weights_out2, acc3)


# speedup vs baseline: 1.4465x; 1.4465x over previous
import functools

import jax
import jax.numpy as jnp
from jax.experimental import pallas as pl
from jax.experimental.pallas import tpu as pltpu

N_FEATURES = 4096
IN_FEATURES = 4096
OUT_FEATURES = 4096
BATCH = 4096
REPLACE_RATE = 1e-4
DECAY_RATE = 0.99
MATURITY_THRESHOLD = 100

_BR = 256
_K = 8


def _stats_body(wout_ref, x_ref, wout_copy_ref, wsum_ref, xsum_ref):
    i = pl.program_id(0)
    wout = wout_ref[...]
    x = x_ref[...]
    wpart = jnp.sum(jnp.abs(wout), axis=0, keepdims=True)
    xpart = jnp.sum(jnp.abs(x), axis=0, keepdims=True)

    @pl.when(i == 0)
    def _():
        wsum_ref[...] = wpart
        xsum_ref[...] = xpart

    @pl.when(i != 0)
    def _():
        wsum_ref[...] += wpart
        xsum_ref[...] += xpart

    wout_copy_ref[...] = wout


def _stats_pass(weights_out, input_values):
    grid = (OUT_FEATURES // _BR,)
    return pl.pallas_call(
        _stats_body,
        grid=grid,
        in_specs=[
            pl.BlockSpec((_BR, N_FEATURES), lambda i: (i, 0)),
            pl.BlockSpec((_BR, N_FEATURES), lambda i: (i, 0)),
        ],
        out_specs=[
            pl.BlockSpec((_BR, N_FEATURES), lambda i: (i, 0)),
            pl.BlockSpec((1, N_FEATURES), lambda i: (0, 0)),
            pl.BlockSpec((1, N_FEATURES), lambda i: (0, 0)),
        ],
        out_shape=[
            jax.ShapeDtypeStruct((OUT_FEATURES, N_FEATURES), jnp.float32),
            jax.ShapeDtypeStruct((1, N_FEATURES), jnp.float32),
            jax.ShapeDtypeStruct((1, N_FEATURES), jnp.float32),
        ],
    )(weights_out, input_values)


def _order_stat_bits(bits, k):
    def body(_, carry):
        lo, hi = carry
        mid = jax.lax.div(lo + hi, 2)
        cnt = jnp.sum((bits <= mid).astype(jnp.int32))
        big = cnt >= k
        return jnp.where(big, lo, mid + 1), jnp.where(big, mid, hi)

    lo = jnp.int32(0)
    hi = jnp.max(bits)
    lo, hi = jax.lax.fori_loop(0, 31, body, (lo, hi))
    return hi


def _rank_body(wsum_ref, xsum_ref, util_ref, age_ref, noise_ref, acc_ref,
               util3_ref, age3_ref, mask_ref, nrepl_ref, sidx_ref, svalid_ref):
    wsum = wsum_ref[...]
    xsum = xsum_ref[...]
    utility = util_ref[...]
    age = age_ref[...]
    noise = noise_ref[...]

    age2 = age + 1
    input_magnitudes = xsum * jnp.float32(1.0 / BATCH)
    step_utility = input_magnitudes * wsum
    utility2 = (jnp.float32(1.0 - DECAY_RATE) * step_utility
                + jnp.float32(DECAY_RATE) * utility)

    acc2 = acc_ref[0] + jnp.float32(REPLACE_RATE * N_FEATURES)
    n_available = acc2.astype(jnp.int32)

    eligibility = age2 > MATURITY_THRESHOLD
    n_eligible = jnp.sum(eligibility.astype(jnp.int32))
    n_repl = jnp.where(n_available > 0,
                       jnp.minimum(n_available, n_eligible),
                       0)

    perturbed = utility2 + noise
    inf = jnp.float32(jnp.inf)
    filtered = jnp.where(eligibility, perturbed, inf)

    m1 = jnp.min(filtered)
    c1 = jnp.sum((filtered == m1).astype(jnp.int32))
    m2 = jnp.min(jnp.where(filtered > m1, filtered, inf))
    threshold = jnp.where(c1 >= n_repl, m1, m2)

    prune = (filtered <= threshold) & eligibility & (n_repl > 0)

    bits = jax.lax.bitcast_convert_type(utility2, jnp.int32)
    half = N_FEATURES // 2
    b_lo = _order_stat_bits(bits, half)
    b_hi = _order_stat_bits(bits, half + 1)
    s_lo = jax.lax.bitcast_convert_type(b_lo, jnp.float32)
    s_hi = jax.lax.bitcast_convert_type(b_hi, jnp.float32)
    median = (s_lo + s_hi) * jnp.float32(0.5)

    util3_ref[...] = jnp.where(prune, median, utility2)
    age3_ref[...] = jnp.where(prune, 0, age2)
    mask_ref[...] = prune.astype(jnp.int32)
    nrepl_ref[0] = n_repl

    positions = jax.lax.broadcasted_iota(jnp.int32, (1, N_FEATURES), 1)
    big = jnp.int32(N_FEATURES)
    pad = jnp.min(jnp.where(prune, big, positions))

    def slot(k, posm):
        m = jnp.min(posm)
        found = m < big
        sidx_ref[k] = jnp.where(found, m, pad)
        svalid_ref[k] = found.astype(jnp.int32)
        return jnp.where(posm == m, big, posm)

    jax.lax.fori_loop(0, _K, slot, jnp.where(prune, positions, big))


def _rank_pass(wsum, xsum, utility, age, noise, acc_in):
    outs = pl.pallas_call(
        _rank_body,
        in_specs=[
            pl.BlockSpec((1, N_FEATURES), lambda: (0, 0)),
            pl.BlockSpec((1, N_FEATURES), lambda: (0, 0)),
            pl.BlockSpec((1, N_FEATURES), lambda: (0, 0)),
            pl.BlockSpec((1, N_FEATURES), lambda: (0, 0)),
            pl.BlockSpec((1, N_FEATURES), lambda: (0, 0)),
            pl.BlockSpec(memory_space=pltpu.SMEM),
        ],
        out_specs=[
            pl.BlockSpec((1, N_FEATURES), lambda: (0, 0)),
            pl.BlockSpec((1, N_FEATURES), lambda: (0, 0)),
            pl.BlockSpec((1, N_FEATURES), lambda: (0, 0)),
            pl.BlockSpec(memory_space=pltpu.SMEM),
            pl.BlockSpec(memory_space=pltpu.SMEM),
            pl.BlockSpec(memory_space=pltpu.SMEM),
        ],
        out_shape=[
            jax.ShapeDtypeStruct((1, N_FEATURES), jnp.float32),
            jax.ShapeDtypeStruct((1, N_FEATURES), jnp.int32),
            jax.ShapeDtypeStruct((1, N_FEATURES), jnp.int32),
            jax.ShapeDtypeStruct((1,), jnp.int32),
            jax.ShapeDtypeStruct((_K,), jnp.int32),
            jax.ShapeDtypeStruct((_K,), jnp.int32),
        ],
    )(wsum, xsum, utility, age, noise, acc_in)
    return outs


_LECUN_LIMIT = 0.027063293382525444


def _rotl(x, r):
    return jax.lax.shift_left(x, jnp.int32(r)) | jax.lax.shift_right_logical(
        x, jnp.int32(32 - r))


def _threefry_round(x0, x1, r):
    x0 = x0 + x1
    x1 = _rotl(x1, r)
    x1 = x0 ^ x1
    return x0, x1


def _lecun_uniform_bits(fcnt, k0, k1):
    ks2 = k0 ^ k1 ^ jnp.int32(0x1BD11BDA)
    x0 = jnp.zeros(fcnt.shape, jnp.int32) + k0
    x1 = fcnt + k1

    r1 = (13, 15, 26, 6)
    r2 = (17, 29, 16, 24)
    for r in r1:
        x0, x1 = _threefry_round(x0, x1, r)
    x0 = x0 + k1
    x1 = x1 + ks2 + jnp.int32(1)
    for r in r2:
        x0, x1 = _threefry_round(x0, x1, r)
    x0 = x0 + ks2
    x1 = x1 + k0 + jnp.int32(2)
    for r in r1:
        x0, x1 = _threefry_round(x0, x1, r)
    x0 = x0 + k0
    x1 = x1 + k1 + jnp.int32(3)
    for r in r2:
        x0, x1 = _threefry_round(x0, x1, r)
    x0 = x0 + k1
    x1 = x1 + ks2 + jnp.int32(4)
    for r in r1:
        x0, x1 = _threefry_round(x0, x1, r)
    x0 = x0 + ks2
    x1 = x1 + k0 + jnp.int32(5)

    bits = x0 ^ x1
    float_bits = jax.lax.shift_right_logical(bits, jnp.int32(9)) | jnp.int32(
        0x3F800000)
    floats = jax.lax.bitcast_convert_type(float_bits, jnp.float32) - jnp.float32(1.0)
    mn = jnp.float32(-_LECUN_LIMIT)
    mx = jnp.float32(_LECUN_LIMIT)
    return jnp.maximum(mn, floats * (mx - mn) + mn)


def _win_body(skey_ref, win_ref, maskc_ref, out_ref):
    i = pl.program_id(0)
    maskc = maskc_ref[...]
    has_pruned = jnp.sum(maskc) > 0

    @pl.when(jnp.logical_not(has_pruned))
    def _():
        out_ref[...] = win_ref[...]

    @pl.when(has_pruned)
    def _():
        rows = (jax.lax.broadcasted_iota(jnp.int32, (_BR, IN_FEATURES), 0)
                + i * _BR)
        cols = jax.lax.broadcasted_iota(jnp.int32, (_BR, IN_FEATURES), 1)
        fcnt = rows * jnp.int32(IN_FEATURES) + cols
        fresh = _lecun_uniform_bits(fcnt, skey_ref[0], skey_ref[1])
        out_ref[...] = jnp.where(maskc != 0, fresh, win_ref[...])


def _win_pass(skey, weights_in, maskcol):
    grid_spec = pltpu.PrefetchScalarGridSpec(
        num_scalar_prefetch=1,
        grid=(N_FEATURES // _BR,),
        in_specs=[
            pl.BlockSpec((_BR, IN_FEATURES), lambda i, kd: (i, 0)),
            pl.BlockSpec((_BR, 1), lambda i, kd: (i, 0)),
        ],
        out_specs=pl.BlockSpec((_BR, IN_FEATURES), lambda i, kd: (i, 0)),
    )
    return pl.pallas_call(
        _win_body,
        grid_spec=grid_spec,
        out_shape=jax.ShapeDtypeStruct((N_FEATURES, IN_FEATURES),
                                       jnp.float32),
    )(skey, weights_in, maskcol)


def _colzero_body(sidx_ref, maskg_ref, woutg_ref, wout_out_ref):
    m = maskg_ref[...].reshape(1, 128) != 0
    wout_out_ref[...] = jnp.where(m, jnp.float32(0.0), woutg_ref[...])


def _colzero_pass(sidx, mask3, wout_copy):
    grid_spec = pltpu.PrefetchScalarGridSpec(
        num_scalar_prefetch=1,
        grid=(_K,),
        in_specs=[
            pl.BlockSpec((1, 1, 128),
                         lambda i, s: (s[i] // 128, 0, 0)),
            pl.BlockSpec((OUT_FEATURES, 128),
                         lambda i, s: (0, s[i] // 128)),
        ],
        out_specs=pl.BlockSpec((OUT_FEATURES, 128),
                               lambda i, s: (0, s[i] // 128)),
    )
    return pl.pallas_call(
        _colzero_body,
        grid_spec=grid_spec,
        out_shape=jax.ShapeDtypeStruct((OUT_FEATURES, N_FEATURES),
                                       jnp.float32),
        input_output_aliases={2: 0},
    )(sidx, mask3, wout_copy)


def kernel(weights_in, weights_out, input_values, age, utility,
           replacement_accumulator):
    noise_key, in_key = jax.random.split(jax.random.key(42))
    noise = jax.random.normal(noise_key, (N_FEATURES,)) * 1e-12
    skey = jax.random.key_data(in_key).astype(jnp.uint32).view(jnp.int32)

    wout_copy, wsum, xsum = _stats_pass(weights_out, input_values)

    acc2 = replacement_accumulator + REPLACE_RATE * N_FEATURES
    util3, age3, mask_i32, nrepl, sidx, svalid = _rank_pass(
        wsum, xsum,
        utility.reshape(1, N_FEATURES),
        age.reshape(1, N_FEATURES),
        noise.reshape(1, N_FEATURES),
        replacement_accumulator.reshape(1),
    )
    mask_flat = mask_i32.reshape(N_FEATURES)
    prune_mask = mask_flat.astype(jnp.bool_)
    n_repl = nrepl[0]

    weights_in2 = _win_pass(skey, weights_in,
                            mask_flat.reshape(N_FEATURES, 1))
    weights_out2 = _colzero_pass(
        sidx, mask_i32.reshape(N_FEATURES // 128, 1, 128), wout_copy)

    utility3 = util3.reshape(N_FEATURES)
    age3 = age3.reshape(N_FEATURES)
    acc3 = acc2 - n_repl.astype(jnp.float32)

    return (utility3, age3, prune_mask, weights_in2, weights_out2, acc3)


# speedup vs baseline: 1.4939x; 1.0328x over previous
import functools

import jax
import jax.numpy as jnp
from jax.experimental import pallas as pl
from jax.experimental.pallas import tpu as pltpu

N_FEATURES = 4096
IN_FEATURES = 4096
OUT_FEATURES = 4096
BATCH = 4096
REPLACE_RATE = 1e-4
DECAY_RATE = 0.99
MATURITY_THRESHOLD = 100

_BR = 256
_K = 8


def _stats_body(wout_ref, x_ref, wout_copy_ref, wsum_ref, xsum_ref):
    i = pl.program_id(0)
    wout = wout_ref[...]
    x = x_ref[...]
    wpart = jnp.sum(jnp.abs(wout), axis=0, keepdims=True)
    xpart = jnp.sum(jnp.abs(x), axis=0, keepdims=True)

    @pl.when(i == 0)
    def _():
        wsum_ref[...] = wpart
        xsum_ref[...] = xpart

    @pl.when(i != 0)
    def _():
        wsum_ref[...] += wpart
        xsum_ref[...] += xpart

    wout_copy_ref[...] = wout


def _stats_pass(weights_out, input_values):
    grid = (OUT_FEATURES // _BR,)
    return pl.pallas_call(
        _stats_body,
        grid=grid,
        in_specs=[
            pl.BlockSpec((_BR, N_FEATURES), lambda i: (i, 0)),
            pl.BlockSpec((_BR, N_FEATURES), lambda i: (i, 0)),
        ],
        out_specs=[
            pl.BlockSpec((_BR, N_FEATURES), lambda i: (i, 0)),
            pl.BlockSpec((1, N_FEATURES), lambda i: (0, 0)),
            pl.BlockSpec((1, N_FEATURES), lambda i: (0, 0)),
        ],
        out_shape=[
            jax.ShapeDtypeStruct((OUT_FEATURES, N_FEATURES), jnp.float32),
            jax.ShapeDtypeStruct((1, N_FEATURES), jnp.float32),
            jax.ShapeDtypeStruct((1, N_FEATURES), jnp.float32),
        ],
    )(weights_out, input_values)


def _order_stat_bits2(bits, k1, k2):
    def body(_, carry):
        lo1, hi1, lo2, hi2 = carry
        mid1 = jax.lax.div(lo1 + hi1, 2)
        mid2 = jax.lax.div(lo2 + hi2, 2)
        cnt1 = jnp.sum((bits <= mid1).astype(jnp.int32))
        cnt2 = jnp.sum((bits <= mid2).astype(jnp.int32))
        big1 = cnt1 >= k1
        big2 = cnt2 >= k2
        return (jnp.where(big1, lo1, mid1 + 1), jnp.where(big1, mid1, hi1),
                jnp.where(big2, lo2, mid2 + 1), jnp.where(big2, mid2, hi2))

    lo = jnp.int32(0)
    hi = jnp.max(bits)
    _, h1, _, h2 = jax.lax.fori_loop(0, 31, body, (lo, hi, lo, hi))
    return h1, h2


def _rank_body(wsum_ref, xsum_ref, util_ref, age_ref, noise_ref, acc_ref,
               util3_ref, age3_ref, mask_ref, nrepl_ref, sidx_ref, svalid_ref):
    wsum = wsum_ref[...]
    xsum = xsum_ref[...]
    utility = util_ref[...]
    age = age_ref[...]
    noise = noise_ref[...]

    age2 = age + 1
    input_magnitudes = xsum * jnp.float32(1.0 / BATCH)
    step_utility = input_magnitudes * wsum
    utility2 = (jnp.float32(1.0 - DECAY_RATE) * step_utility
                + jnp.float32(DECAY_RATE) * utility)

    acc2 = acc_ref[0] + jnp.float32(REPLACE_RATE * N_FEATURES)
    n_available = acc2.astype(jnp.int32)

    eligibility = age2 > MATURITY_THRESHOLD
    n_eligible = jnp.sum(eligibility.astype(jnp.int32))
    n_repl = jnp.where(n_available > 0,
                       jnp.minimum(n_available, n_eligible),
                       0)

    perturbed = utility2 + noise
    inf = jnp.float32(jnp.inf)
    filtered = jnp.where(eligibility, perturbed, inf)

    m1 = jnp.min(filtered)
    c1 = jnp.sum((filtered == m1).astype(jnp.int32))
    m2 = jnp.min(jnp.where(filtered > m1, filtered, inf))
    threshold = jnp.where(c1 >= n_repl, m1, m2)

    prune = (filtered <= threshold) & eligibility & (n_repl > 0)

    bits = jax.lax.bitcast_convert_type(utility2, jnp.int32)
    half = N_FEATURES // 2
    b_lo, b_hi = _order_stat_bits2(bits, half, half + 1)
    s_lo = jax.lax.bitcast_convert_type(b_lo, jnp.float32)
    s_hi = jax.lax.bitcast_convert_type(b_hi, jnp.float32)
    median = (s_lo + s_hi) * jnp.float32(0.5)

    util3_ref[...] = jnp.where(prune, median, utility2)
    age3_ref[...] = jnp.where(prune, 0, age2)
    mask_ref[...] = prune.astype(jnp.int32)
    nrepl_ref[0] = n_repl

    positions = jax.lax.broadcasted_iota(jnp.int32, (1, N_FEATURES), 1)
    big = jnp.int32(N_FEATURES)
    pad = jnp.min(jnp.where(prune, big, positions))

    def slot(k, posm):
        m = jnp.min(posm)
        found = m < big
        sidx_ref[k] = jnp.where(found, m, pad)
        svalid_ref[k] = found.astype(jnp.int32)
        return jnp.where(posm == m, big, posm)

    jax.lax.fori_loop(0, _K, slot, jnp.where(prune, positions, big))


def _rank_pass(wsum, xsum, utility, age, noise, acc_in):
    outs = pl.pallas_call(
        _rank_body,
        in_specs=[
            pl.BlockSpec((1, N_FEATURES), lambda: (0, 0)),
            pl.BlockSpec((1, N_FEATURES), lambda: (0, 0)),
            pl.BlockSpec((1, N_FEATURES), lambda: (0, 0)),
            pl.BlockSpec((1, N_FEATURES), lambda: (0, 0)),
            pl.BlockSpec((1, N_FEATURES), lambda: (0, 0)),
            pl.BlockSpec(memory_space=pltpu.SMEM),
        ],
        out_specs=[
            pl.BlockSpec((1, N_FEATURES), lambda: (0, 0)),
            pl.BlockSpec((1, N_FEATURES), lambda: (0, 0)),
            pl.BlockSpec((1, N_FEATURES), lambda: (0, 0)),
            pl.BlockSpec(memory_space=pltpu.SMEM),
            pl.BlockSpec(memory_space=pltpu.SMEM),
            pl.BlockSpec(memory_space=pltpu.SMEM),
        ],
        out_shape=[
            jax.ShapeDtypeStruct((1, N_FEATURES), jnp.float32),
            jax.ShapeDtypeStruct((1, N_FEATURES), jnp.int32),
            jax.ShapeDtypeStruct((1, N_FEATURES), jnp.int32),
            jax.ShapeDtypeStruct((1,), jnp.int32),
            jax.ShapeDtypeStruct((_K,), jnp.int32),
            jax.ShapeDtypeStruct((_K,), jnp.int32),
        ],
    )(wsum, xsum, utility, age, noise, acc_in)
    return outs


_LECUN_LIMIT = 0.027063293382525444


def _rotl(x, r):
    return jax.lax.shift_left(x, jnp.int32(r)) | jax.lax.shift_right_logical(
        x, jnp.int32(32 - r))


def _threefry_round(x0, x1, r):
    x0 = x0 + x1
    x1 = _rotl(x1, r)
    x1 = x0 ^ x1
    return x0, x1


def _lecun_uniform_bits(fcnt, k0, k1):
    ks2 = k0 ^ k1 ^ jnp.int32(0x1BD11BDA)
    x0 = jnp.zeros(fcnt.shape, jnp.int32) + k0
    x1 = fcnt + k1

    r1 = (13, 15, 26, 6)
    r2 = (17, 29, 16, 24)
    for r in r1:
        x0, x1 = _threefry_round(x0, x1, r)
    x0 = x0 + k1
    x1 = x1 + ks2 + jnp.int32(1)
    for r in r2:
        x0, x1 = _threefry_round(x0, x1, r)
    x0 = x0 + ks2
    x1 = x1 + k0 + jnp.int32(2)
    for r in r1:
        x0, x1 = _threefry_round(x0, x1, r)
    x0 = x0 + k0
    x1 = x1 + k1 + jnp.int32(3)
    for r in r2:
        x0, x1 = _threefry_round(x0, x1, r)
    x0 = x0 + k1
    x1 = x1 + ks2 + jnp.int32(4)
    for r in r1:
        x0, x1 = _threefry_round(x0, x1, r)
    x0 = x0 + ks2
    x1 = x1 + k0 + jnp.int32(5)

    bits = x0 ^ x1
    float_bits = jax.lax.shift_right_logical(bits, jnp.int32(9)) | jnp.int32(
        0x3F800000)
    floats = jax.lax.bitcast_convert_type(float_bits, jnp.float32) - jnp.float32(1.0)
    mn = jnp.float32(-_LECUN_LIMIT)
    mx = jnp.float32(_LECUN_LIMIT)
    return jnp.maximum(mn, floats * (mx - mn) + mn)


def _win_body(skey_ref, win_ref, maskc_ref, out_ref):
    i = pl.program_id(0)
    maskc = maskc_ref[...]
    has_pruned = jnp.sum(maskc) > 0

    @pl.when(jnp.logical_not(has_pruned))
    def _():
        out_ref[...] = win_ref[...]

    @pl.when(has_pruned)
    def _():
        rows = (jax.lax.broadcasted_iota(jnp.int32, (_BR, IN_FEATURES), 0)
                + i * _BR)
        cols = jax.lax.broadcasted_iota(jnp.int32, (_BR, IN_FEATURES), 1)
        fcnt = rows * jnp.int32(IN_FEATURES) + cols
        fresh = _lecun_uniform_bits(fcnt, skey_ref[0], skey_ref[1])
        out_ref[...] = jnp.where(maskc != 0, fresh, win_ref[...])


def _win_pass(skey, weights_in, maskcol):
    grid_spec = pltpu.PrefetchScalarGridSpec(
        num_scalar_prefetch=1,
        grid=(N_FEATURES // _BR,),
        in_specs=[
            pl.BlockSpec((_BR, IN_FEATURES), lambda i, kd: (i, 0)),
            pl.BlockSpec((_BR, 1), lambda i, kd: (i, 0)),
        ],
        out_specs=pl.BlockSpec((_BR, IN_FEATURES), lambda i, kd: (i, 0)),
    )
    return pl.pallas_call(
        _win_body,
        grid_spec=grid_spec,
        out_shape=jax.ShapeDtypeStruct((N_FEATURES, IN_FEATURES),
                                       jnp.float32),
    )(skey, weights_in, maskcol)


def _colzero_body(sidx_ref, maskg_ref, woutg_ref, wout_out_ref):
    m = maskg_ref[...].reshape(1, 128) != 0
    wout_out_ref[...] = jnp.where(m, jnp.float32(0.0), woutg_ref[...])


def _colzero_pass(sidx, mask3, wout_copy):
    grid_spec = pltpu.PrefetchScalarGridSpec(
        num_scalar_prefetch=1,
        grid=(_K,),
        in_specs=[
            pl.BlockSpec((1, 1, 128),
                         lambda i, s: (s[i] // 128, 0, 0)),
            pl.BlockSpec((OUT_FEATURES, 128),
                         lambda i, s: (0, s[i] // 128)),
        ],
        out_specs=pl.BlockSpec((OUT_FEATURES, 128),
                               lambda i, s: (0, s[i] // 128)),
    )
    return pl.pallas_call(
        _colzero_body,
        grid_spec=grid_spec,
        out_shape=jax.ShapeDtypeStruct((OUT_FEATURES, N_FEATURES),
                                       jnp.float32),
        input_output_aliases={2: 0},
    )(sidx, mask3, wout_copy)


def kernel(weights_in, weights_out, input_values, age, utility,
           replacement_accumulator):
    noise_key, in_key = jax.random.split(jax.random.key(42))
    noise = jax.random.normal(noise_key, (N_FEATURES,)) * 1e-12
    skey = jax.random.key_data(in_key).astype(jnp.uint32).view(jnp.int32)

    wout_copy, wsum, xsum = _stats_pass(weights_out, input_values)

    acc2 = replacement_accumulator + REPLACE_RATE * N_FEATURES
    util3, age3, mask_i32, nrepl, sidx, svalid = _rank_pass(
        wsum, xsum,
        utility.reshape(1, N_FEATURES),
        age.reshape(1, N_FEATURES),
        noise.reshape(1, N_FEATURES),
        replacement_accumulator.reshape(1),
    )
    mask_flat = mask_i32.reshape(N_FEATURES)
    prune_mask = mask_flat.astype(jnp.bool_)
    n_repl = nrepl[0]

    weights_in2 = _win_pass(skey, weights_in,
                            mask_flat.reshape(N_FEATURES, 1))
    weights_out2 = _colzero_pass(
        sidx, mask_i32.reshape(N_FEATURES // 128, 1, 128), wout_copy)

    utility3 = util3.reshape(N_FEATURES)
    age3 = age3.reshape(N_FEATURES)
    acc3 = acc2 - n_repl.astype(jnp.float32)

    return (utility3, age3, prune_mask, weights_in2, weights_out2, acc3)
